# Initial kernel scaffold; baseline (speedup 1.0000x reference)
import jax
import jax.numpy as jnp
from jax.experimental import pallas as pl


def kernel(input_labels, pos_labels, neg_labels, in_embed, out_embed):
    raise NotImplementedError("write your pallas kernel here")


# speedup vs baseline: 3.1957x; 3.1957x over previous
import functools

import jax
import jax.numpy as jnp
from jax import lax
from jax.experimental import pallas as pl
from jax.experimental.pallas import tpu as pltpu
from jax.experimental.pallas import tpu_sc as plsc

_B = 16384
_D = 64
_POS = 10
_NEG = 50
_R = _POS + _NEG

_NC = 2
_NS = 16
_NW = _NC * _NS
_BPW = _B // _NW
_C = 8
_NIT = _BPW // _C
_CR = _C * _R
_G = 120
_NG = _CR // _G


def _sc_dots(in_embed, out_embed, input_labels, out_idx_flat):
  mesh = plsc.VectorSubcoreMesh(core_axis_name="c", subcore_axis_name="s")

  @functools.partial(
      pl.kernel,
      out_type=jax.ShapeDtypeStruct((_B * 64,), jnp.float32),
      mesh=mesh,
      compiler_params=pltpu.CompilerParams(use_tc_tiling_on_sc=False),
      scratch_types=[
          pltpu.VMEM((_C,), jnp.int32),
          pltpu.VMEM((_CR,), jnp.int32),
          pltpu.VMEM((_C, _D), jnp.float32),
          pltpu.VMEM((_CR + 8, _D), jnp.float32),
          pltpu.VMEM((_C * 64,), jnp.float32),
          pltpu.SemaphoreType.DMA,
      ],
  )
  def sc_kernel(in_tab, out_tab, in_lab, out_idx, dots_hbm,
                iidx_v, oidx_v, irows_v, orows_v, dots_v, sem):
    wid = lax.axis_index("s") * _NC + lax.axis_index("c")

    def body(i, carry):
      base = wid * _BPW + i * _C
      pltpu.sync_copy(in_lab.at[pl.ds(base, _C)], iidx_v)
      pltpu.sync_copy(out_idx.at[pl.ds(base * _R, _CR)], oidx_v)
      pltpu.async_copy(in_tab.at[iidx_v], irows_v, sem).wait()
      for g in range(_NG):
        pltpu.async_copy(
            out_tab.at[oidx_v.at[pl.ds(g * _G, _G)]],
            orows_v.at[pl.ds(g * _G, _G)], sem).wait()

      def group_body(eg, carry2):
        e = eg // 4
        g = eg - e * 4
        inp = [irows_v[e, pl.ds(q * 16, 16)] for q in range(4)]
        base_row = e * _R + g * 16
        lane = lax.iota(jnp.int32, 16)

        accs = []
        for k in range(16):
          row = base_row + k
          acc = orows_v[row, pl.ds(0, 16)] * inp[0]
          for q in range(1, 4):
            acc = acc + orows_v[row, pl.ds(q * 16, 16)] * inp[q]
          accs.append(acc)

        def merge(a, b, s):
          idx = lane ^ s
          sa = a + a.at[idx].get(mode="promise_in_bounds")
          sb = b + b.at[idx].get(mode="promise_in_bounds")
          return jnp.where((lane & s) == 0, sa, sb)

        lvl = accs
        for s in (1, 2, 4, 8):
          lvl = [merge(lvl[2 * i], lvl[2 * i + 1], s)
                 for i in range(len(lvl) // 2)]
        dots_v[pl.ds(e * 64 + g * 16, 16)] = lvl[0]
        return carry2

      lax.fori_loop(0, _C * 4, group_body, 0)

      pltpu.sync_copy(dots_v, dots_hbm.at[pl.ds(base * 64, _C * 64)])
      return carry

    lax.fori_loop(0, _NIT, body, 0)

  return sc_kernel(in_embed, out_embed, input_labels, out_idx_flat)


def _tc_loss(dots):
  blk = 512
  grid = _B // blk

  def tc_body(dots_ref, out_ref):
    x = dots_ref[...]
    col = lax.broadcasted_iota(jnp.int32, x.shape, 1)
    xs = jnp.where(col < _POS, x, -x)
    ls = jnp.minimum(xs, 0.0) - jnp.log1p(jnp.exp(-jnp.abs(xs)))
    ls = jnp.where(col < _R, ls, 0.0)
    out_ref[...] = -jnp.sum(ls, axis=1)

  return pl.pallas_call(
      tc_body,
      grid=(grid,),
      in_specs=[pl.BlockSpec((blk, 64), lambda i: (i, 0))],
      out_specs=pl.BlockSpec((blk,), lambda i: (i,)),
      out_shape=jax.ShapeDtypeStruct((_B,), jnp.float32),
  )(dots)


def kernel(input_labels, pos_labels, neg_labels, in_embed, out_embed):
  out_idx = jnp.concatenate([pos_labels, neg_labels], axis=1).reshape(-1)
  dots = _sc_dots(in_embed, out_embed, input_labels, out_idx)
  return _tc_loss(dots.reshape(_B, 64))


# speedup vs baseline: 4.0229x; 1.2589x over previous
import functools

import jax
import jax.numpy as jnp
from jax import lax
from jax.experimental import pallas as pl
from jax.experimental.pallas import tpu as pltpu
from jax.experimental.pallas import tpu_sc as plsc

_B = 16384
_D = 64
_POS = 10
_NEG = 50
_R = _POS + _NEG

_NC = 2
_NS = 16
_NW = _NC * _NS
_BPW = _B // _NW
_C = 8
_NIT = _BPW // _C
_CR = _C * _R
_G = 120
_NG = _CR // _G
_ORB = _CR + 8


def _sc_dots(in_embed, out_embed, input_labels, out_idx_flat):
  mesh = plsc.VectorSubcoreMesh(core_axis_name="c", subcore_axis_name="s")

  @functools.partial(
      pl.kernel,
      out_type=jax.ShapeDtypeStruct((_B * 64,), jnp.float32),
      mesh=mesh,
      compiler_params=pltpu.CompilerParams(use_tc_tiling_on_sc=False),
      scratch_types=[
          pltpu.VMEM((2, _C), jnp.int32),
          pltpu.VMEM((2, _CR), jnp.int32),
          pltpu.VMEM((2, _C, _D), jnp.float32),
          pltpu.VMEM((2, _ORB, _D), jnp.float32),
          pltpu.VMEM((2, _C * 64), jnp.float32),
          pltpu.SemaphoreType.DMA,
          pltpu.SemaphoreType.DMA,
          pltpu.SemaphoreType.DMA,
          pltpu.SemaphoreType.DMA,
          pltpu.SemaphoreType.DMA,
          pltpu.SemaphoreType.DMA,
      ],
  )
  def sc_kernel(in_tab, out_tab, in_lab, out_idx, dots_hbm,
                iidx_v, oidx_v, irows_v, orows_v, dots_v,
                sidx0, sidx1, sgat0, sgat1, sdot0, sdot1):
    wid = lax.axis_index("s") * _NC + lax.axis_index("c")
    sidx = (sidx0, sidx1)
    sgat = (sgat0, sgat1)
    sdot = (sdot0, sdot1)

    def issue_idx(k, slot):
      base = wid * _BPW + k * _C
      pltpu.make_async_copy(
          in_lab.at[pl.ds(base, _C)], iidx_v.at[slot], sidx[slot]).start()
      pltpu.make_async_copy(
          out_idx.at[pl.ds(base * _R, _CR)], oidx_v.at[slot],
          sidx[slot]).start()

    def wait_idx(slot):
      pltpu.make_async_copy(
          in_lab.at[pl.ds(0, _C)], iidx_v.at[slot], sidx[slot]).wait()
      pltpu.make_async_copy(
          out_idx.at[pl.ds(0, _CR)], oidx_v.at[slot], sidx[slot]).wait()

    def issue_gat(slot):
      pltpu.make_async_copy(
          in_tab.at[iidx_v.at[slot]], irows_v.at[slot], sgat[slot]).start()
      for g in range(_NG):
        pltpu.make_async_copy(
            out_tab.at[oidx_v.at[slot, pl.ds(g * _G, _G)]],
            orows_v.at[slot, pl.ds(g * _G, _G)], sgat[slot]).start()

    def wait_gat(slot):
      pltpu.make_async_copy(
          in_tab.at[iidx_v.at[slot]], irows_v.at[slot], sgat[slot]).wait()
      for g in range(_NG):
        pltpu.make_async_copy(
            out_tab.at[oidx_v.at[slot, pl.ds(g * _G, _G)]],
            orows_v.at[slot, pl.ds(g * _G, _G)], sgat[slot]).wait()

    def issue_dots(k, slot):
      base = wid * _BPW + k * _C
      pltpu.make_async_copy(
          dots_v.at[slot], dots_hbm.at[pl.ds(base * 64, _C * 64)],
          sdot[slot]).start()

    def wait_dots(slot):
      pltpu.make_async_copy(
          dots_v.at[slot], dots_hbm.at[pl.ds(0, _C * 64)],
          sdot[slot]).wait()

    def compute(slot):
      lane = lax.iota(jnp.int32, 16)

      def group_body(eg, carry2):
        e = eg // 4
        g = eg - e * 4
        inp = [irows_v[slot, e, pl.ds(q * 16, 16)] for q in range(4)]
        base_row = e * _R + g * 16

        accs = []
        for kk in range(16):
          row = base_row + kk
          acc = orows_v[slot, row, pl.ds(0, 16)] * inp[0]
          for q in range(1, 4):
            acc = acc + orows_v[slot, row, pl.ds(q * 16, 16)] * inp[q]
          accs.append(acc)

        def merge(a, b, s):
          idx = lane ^ s
          sa = a + a.at[idx].get(mode="promise_in_bounds")
          sb = b + b.at[idx].get(mode="promise_in_bounds")
          return jnp.where((lane & s) == 0, sa, sb)

        lvl = accs
        for s in (1, 2, 4, 8):
          lvl = [merge(lvl[2 * i], lvl[2 * i + 1], s)
                 for i in range(len(lvl) // 2)]
        dots_v[slot, pl.ds(e * 64 + g * 16, 16)] = lvl[0]
        return carry2

      lax.fori_loop(0, _C * 4, group_body, 0)

    issue_idx(0, 0)
    issue_idx(1, 1)
    wait_idx(0)
    issue_gat(0)

    def half_body(k, slot):
      wait_gat(slot)

      @pl.when(k + 2 < _NIT)
      def _():
        issue_idx(k + 2, slot)

      @pl.when(k + 1 < _NIT)
      def _():
        wait_idx(slot ^ 1)
        issue_gat(slot ^ 1)

      @pl.when(k >= 2)
      def _():
        wait_dots(slot)

      compute(slot)
      issue_dots(k, slot)

    def body(ii, carry):
      k = ii * 2
      half_body(k, 0)
      half_body(k + 1, 1)
      return carry

    lax.fori_loop(0, _NIT // 2, body, 0)
    wait_dots(0)
    wait_dots(1)

  return sc_kernel(in_embed, out_embed, input_labels, out_idx_flat)


def _tc_loss(dots):
  blk = 512
  grid = _B // blk

  def tc_body(dots_ref, out_ref):
    x = dots_ref[...]
    col = lax.broadcasted_iota(jnp.int32, x.shape, 1)
    xs = jnp.where(col < _POS, x, -x)
    ls = jnp.minimum(xs, 0.0) - jnp.log1p(jnp.exp(-jnp.abs(xs)))
    ls = jnp.where(col < _R, ls, 0.0)
    out_ref[...] = -jnp.sum(ls, axis=1)

  return pl.pallas_call(
      tc_body,
      grid=(grid,),
      in_specs=[pl.BlockSpec((blk, 64), lambda i: (i, 0))],
      out_specs=pl.BlockSpec((blk,), lambda i: (i,)),
      out_shape=jax.ShapeDtypeStruct((_B,), jnp.float32),
  )(dots)


def kernel(input_labels, pos_labels, neg_labels, in_embed, out_embed):
  out_idx = jnp.concatenate([pos_labels, neg_labels], axis=1).reshape(-1)
  dots = _sc_dots(in_embed, out_embed, input_labels, out_idx)
  return _tc_loss(dots.reshape(_B, 64))


# speedup vs baseline: 4.9932x; 1.2412x over previous
import functools

import jax
import jax.numpy as jnp
from jax import lax
from jax.experimental import pallas as pl
from jax.experimental.pallas import tpu as pltpu
from jax.experimental.pallas import tpu_sc as plsc

_V = 1000000
_B = 16384
_D = 64
_POS = 10
_NEG = 50
_R = _POS + _NEG

_NC = 2
_NS = 16
_NW = _NC * _NS
_BPW = _B // _NW
_C = 8
_NIT = _BPW // _C
_CR = _C * _R
_G = 120
_NG = _CR // _NG if False else _CR // _G
_ORB = _CR + 8


def _sc_dots(out_tab_lin, inp_rows_lin, out_idx_flat):
  mesh = plsc.VectorSubcoreMesh(core_axis_name="c", subcore_axis_name="s")

  @functools.partial(
      pl.kernel,
      out_type=jax.ShapeDtypeStruct((_B * 64,), jnp.float32),
      mesh=mesh,
      compiler_params=pltpu.CompilerParams(use_tc_tiling_on_sc=False),
      scratch_types=[
          pltpu.VMEM((2, _CR), jnp.int32),
          pltpu.VMEM((2, _C, _D), jnp.float32),
          pltpu.VMEM((2, _ORB, _D), jnp.float32),
          pltpu.VMEM((2, _C * 64), jnp.float32),
          pltpu.SemaphoreType.DMA,
          pltpu.SemaphoreType.DMA,
          pltpu.SemaphoreType.DMA,
          pltpu.SemaphoreType.DMA,
          pltpu.SemaphoreType.DMA,
          pltpu.SemaphoreType.DMA,
      ],
  )
  def sc_kernel(out_tab, inp_rows, out_idx, dots_hbm,
                oidx_v, irows_v, orows_v, dots_v,
                sidx0, sidx1, sgat0, sgat1, sdot0, sdot1):
    wid = lax.axis_index("s") * _NC + lax.axis_index("c")
    sidx = (sidx0, sidx1)
    sgat = (sgat0, sgat1)
    sdot = (sdot0, sdot1)

    def issue_idx(k, slot):
      base = wid * _BPW + k * _C
      pltpu.make_async_copy(
          out_idx.at[pl.ds(base * _R, _CR)], oidx_v.at[slot],
          sidx[slot]).start()

    def wait_idx(slot):
      pltpu.make_async_copy(
          out_idx.at[pl.ds(0, _CR)], oidx_v.at[slot], sidx[slot]).wait()

    def issue_gat(k, slot):
      base = wid * _BPW + k * _C
      pltpu.make_async_copy(
          inp_rows.at[pl.ds(base, _C)], irows_v.at[slot],
          sgat[slot]).start()
      for g in range(_NG):
        pltpu.make_async_copy(
            out_tab.at[oidx_v.at[slot, pl.ds(g * _G, _G)]],
            orows_v.at[slot, pl.ds(g * _G, _G)], sgat[slot]).start()

    def wait_gat(slot):
      pltpu.make_async_copy(
          inp_rows.at[pl.ds(0, _C)], irows_v.at[slot], sgat[slot]).wait()
      for g in range(_NG):
        pltpu.make_async_copy(
            out_tab.at[oidx_v.at[slot, pl.ds(g * _G, _G)]],
            orows_v.at[slot, pl.ds(g * _G, _G)], sgat[slot]).wait()

    def issue_dots(k, slot):
      base = wid * _BPW + k * _C
      pltpu.make_async_copy(
          dots_v.at[slot], dots_hbm.at[pl.ds(base * 64, _C * 64)],
          sdot[slot]).start()

    def wait_dots(slot):
      pltpu.make_async_copy(
          dots_v.at[slot], dots_hbm.at[pl.ds(0, _C * 64)],
          sdot[slot]).wait()

    def compute(slot):
      lane = lax.iota(jnp.int32, 16)

      def group_body(eg, carry2):
        e = eg // 4
        g = eg - e * 4
        inp = [irows_v[slot, e, pl.ds(q * 16, 16)] for q in range(4)]
        base_row = e * _R + g * 16

        accs = []
        for kk in range(16):
          row = base_row + kk
          acc = orows_v[slot, row, pl.ds(0, 16)] * inp[0]
          for q in range(1, 4):
            acc = acc + orows_v[slot, row, pl.ds(q * 16, 16)] * inp[q]
          accs.append(acc)

        def merge(a, b, s):
          idx = lane ^ s
          sa = a + a.at[idx].get(mode="promise_in_bounds")
          sb = b + b.at[idx].get(mode="promise_in_bounds")
          return jnp.where((lane & s) == 0, sa, sb)

        lvl = accs
        for s in (1, 2, 4, 8):
          lvl = [merge(lvl[2 * i], lvl[2 * i + 1], s)
                 for i in range(len(lvl) // 2)]
        dots_v[slot, pl.ds(e * 64 + g * 16, 16)] = lvl[0]
        return carry2

      lax.fori_loop(0, _C * 4, group_body, 0)

    issue_idx(0, 0)
    issue_idx(1, 1)
    wait_idx(0)
    issue_gat(0, 0)

    def half_body(k, slot):
      wait_gat(slot)

      @pl.when(k + 2 < _NIT)
      def _():
        issue_idx(k + 2, slot)

      @pl.when(k + 1 < _NIT)
      def _():
        wait_idx(slot ^ 1)
        issue_gat(k + 1, slot ^ 1)

      @pl.when(k >= 2)
      def _():
        wait_dots(slot)

      compute(slot)
      issue_dots(k, slot)

    def body(ii, carry):
      k = ii * 2
      half_body(k, 0)
      half_body(k + 1, 1)
      return carry

    lax.fori_loop(0, _NIT // 2, body, 0)
    wait_dots(0)
    wait_dots(1)

  return sc_kernel(out_tab_lin, inp_rows_lin, out_idx_flat)


def _tc_loss(dots):
  blk = 512
  grid = _B // blk

  def tc_body(dots_ref, out_ref):
    x = dots_ref[...]
    col = lax.broadcasted_iota(jnp.int32, x.shape, 1)
    xs = jnp.where(col < _POS, x, -x)
    ls = jnp.minimum(xs, 0.0) - jnp.log1p(jnp.exp(-jnp.abs(xs)))
    ls = jnp.where(col < _R, ls, 0.0)
    out_ref[...] = -jnp.sum(ls, axis=1)

  return pl.pallas_call(
      tc_body,
      grid=(grid,),
      in_specs=[pl.BlockSpec((blk, 64), lambda i: (i, 0))],
      out_specs=pl.BlockSpec((blk,), lambda i: (i,)),
      out_shape=jax.ShapeDtypeStruct((_B,), jnp.float32),
  )(dots)


def _linearize(x, shape):
  flat = lax.optimization_barrier(jnp.reshape(x, (-1,)))
  return flat.reshape(shape)


def kernel(input_labels, pos_labels, neg_labels, in_embed, out_embed):
  out_idx = jnp.concatenate([pos_labels, neg_labels], axis=1).reshape(-1)
  inp_rows = jnp.take(in_embed, input_labels, axis=0)
  inp_lin = _linearize(inp_rows, (_B, _D))
  out_lin = _linearize(out_embed, (_V, _D))
  dots = _sc_dots(out_lin, inp_lin, out_idx)
  return _tc_loss(dots.reshape(_B, 64))


# speedup vs baseline: 5.0319x; 1.0077x over previous
import functools

import jax
import jax.numpy as jnp
from jax import lax
from jax.experimental import pallas as pl
from jax.experimental.pallas import tpu as pltpu
from jax.experimental.pallas import tpu_sc as plsc

_V = 1000000
_B = 16384
_D = 64
_POS = 10
_NEG = 50
_R = _POS + _NEG

_NC = 2
_NS = 16
_NW = _NC * _NS
_BPW = _B // _NW
_C = 8
_NIT = _BPW // _C
_CR = _C * _R
_G = 120
_NG = _CR // _NG if False else _CR // _G
_ORB = _CR + 8


def _sc_dots(out_tab_lin, inp_rows_lin, out_idx_flat):
  mesh = plsc.VectorSubcoreMesh(core_axis_name="c", subcore_axis_name="s")

  @functools.partial(
      pl.kernel,
      out_type=jax.ShapeDtypeStruct((_B * 64,), jnp.float32),
      mesh=mesh,
      compiler_params=pltpu.CompilerParams(use_tc_tiling_on_sc=False),
      scratch_types=[
          pltpu.VMEM((2, _CR), jnp.int32),
          pltpu.VMEM((2, _C, _D), jnp.float32),
          pltpu.VMEM((2, _ORB, _D), jnp.float32),
          pltpu.VMEM((2, _C * 64), jnp.float32),
          pltpu.SemaphoreType.DMA,
          pltpu.SemaphoreType.DMA,
          pltpu.SemaphoreType.DMA,
          pltpu.SemaphoreType.DMA,
          pltpu.SemaphoreType.DMA,
          pltpu.SemaphoreType.DMA,
      ],
  )
  def sc_kernel(out_tab, inp_rows, out_idx, dots_hbm,
                oidx_v, irows_v, orows_v, dots_v,
                sidx0, sidx1, sgat0, sgat1, sdot0, sdot1):
    wid = lax.axis_index("s") * _NC + lax.axis_index("c")
    sidx = (sidx0, sidx1)
    sgat = (sgat0, sgat1)
    sdot = (sdot0, sdot1)

    def issue_idx(k, slot):
      base = wid * _BPW + k * _C
      pltpu.make_async_copy(
          out_idx.at[pl.ds(base * _R, _CR)], oidx_v.at[slot],
          sidx[slot]).start()

    def wait_idx(slot):
      pltpu.make_async_copy(
          out_idx.at[pl.ds(0, _CR)], oidx_v.at[slot], sidx[slot]).wait()

    def issue_gat(k, slot):
      base = wid * _BPW + k * _C
      pltpu.make_async_copy(
          inp_rows.at[pl.ds(base, _C)], irows_v.at[slot],
          sgat[slot]).start()
      for g in range(_NG):
        pltpu.make_async_copy(
            out_tab.at[oidx_v.at[slot, pl.ds(g * _G, _G)]],
            orows_v.at[slot, pl.ds(g * _G, _G)], sgat[slot]).start()

    def wait_gat(slot):
      pltpu.make_async_copy(
          inp_rows.at[pl.ds(0, _C)], irows_v.at[slot], sgat[slot]).wait()
      for g in range(_NG):
        pltpu.make_async_copy(
            out_tab.at[oidx_v.at[slot, pl.ds(g * _G, _G)]],
            orows_v.at[slot, pl.ds(g * _G, _G)], sgat[slot]).wait()

    def issue_dots(k, slot):
      base = wid * _BPW + k * _C
      pltpu.make_async_copy(
          dots_v.at[slot], dots_hbm.at[pl.ds(base * 64, _C * 64)],
          sdot[slot]).start()

    def wait_dots(slot):
      pltpu.make_async_copy(
          dots_v.at[slot], dots_hbm.at[pl.ds(0, _C * 64)],
          sdot[slot]).wait()

    def compute(slot):
      lane = lax.iota(jnp.int32, 16)

      @plsc.parallel_loop(0, _C * 4, 1, unroll=2)
      def group_body(eg):
        e = eg // 4
        g = eg - e * 4
        inp = [irows_v[slot, e, pl.ds(q * 16, 16)] for q in range(4)]
        base_row = e * _R + g * 16

        accs = []
        for kk in range(16):
          row = base_row + kk
          acc = orows_v[slot, row, pl.ds(0, 16)] * inp[0]
          for q in range(1, 4):
            acc = acc + orows_v[slot, row, pl.ds(q * 16, 16)] * inp[q]
          accs.append(acc)

        def merge(a, b, s):
          idx = lane ^ s
          sa = a + a.at[idx].get(mode="promise_in_bounds")
          sb = b + b.at[idx].get(mode="promise_in_bounds")
          return jnp.where((lane & s) == 0, sa, sb)

        lvl = accs
        for s in (1, 2, 4, 8):
          lvl = [merge(lvl[2 * i], lvl[2 * i + 1], s)
                 for i in range(len(lvl) // 2)]
        dots_v[slot, pl.ds(e * 64 + g * 16, 16)] = lvl[0]

    issue_idx(0, 0)
    issue_idx(1, 1)
    wait_idx(0)
    issue_gat(0, 0)

    def half_body(k, slot):
      wait_gat(slot)

      @pl.when(k + 2 < _NIT)
      def _():
        issue_idx(k + 2, slot)

      @pl.when(k + 1 < _NIT)
      def _():
        wait_idx(slot ^ 1)
        issue_gat(k + 1, slot ^ 1)

      @pl.when(k >= 2)
      def _():
        wait_dots(slot)

      compute(slot)
      issue_dots(k, slot)

    def body(ii, carry):
      k = ii * 2
      half_body(k, 0)
      half_body(k + 1, 1)
      return carry

    lax.fori_loop(0, _NIT // 2, body, 0)
    wait_dots(0)
    wait_dots(1)

  return sc_kernel(out_tab_lin, inp_rows_lin, out_idx_flat)


def _tc_loss(dots):
  blk = 512
  grid = _B // blk

  def tc_body(dots_ref, out_ref):
    x = dots_ref[...]
    col = lax.broadcasted_iota(jnp.int32, x.shape, 1)
    xs = jnp.where(col < _POS, x, -x)
    ls = jnp.minimum(xs, 0.0) - jnp.log1p(jnp.exp(-jnp.abs(xs)))
    ls = jnp.where(col < _R, ls, 0.0)
    out_ref[...] = -jnp.sum(ls, axis=1)

  return pl.pallas_call(
      tc_body,
      grid=(grid,),
      in_specs=[pl.BlockSpec((blk, 64), lambda i: (i, 0))],
      out_specs=pl.BlockSpec((blk,), lambda i: (i,)),
      out_shape=jax.ShapeDtypeStruct((_B,), jnp.float32),
  )(dots)


def _linearize(x, shape):
  flat = lax.optimization_barrier(jnp.reshape(x, (-1,)))
  return flat.reshape(shape)


def kernel(input_labels, pos_labels, neg_labels, in_embed, out_embed):
  out_idx = jnp.concatenate([pos_labels, neg_labels], axis=1).reshape(-1)
  out_lin = _linearize(out_embed, (_V, _D))
  inp_rows = jnp.take(in_embed, input_labels, axis=0)
  inp_lin = _linearize(inp_rows, (_B, _D))
  dots = _sc_dots(out_lin, inp_lin, out_idx)
  return _tc_loss(dots.reshape(_B, 64))
